# baseline (speedup 1.0000x reference)
import functools

import jax
import jax.numpy as jnp
from jax import lax
from jax.experimental import pallas as pl
from jax.experimental.pallas import tpu as pltpu
from jax.experimental.pallas import tpu_sc as plsc

N, E, D, H, DK, DV = 10000, 160000, 256, 8, 64, 64
HDK = H * DK
NC, NS, L = 2, 16, 16
NW = NC * NS
C = E // NW
G = 10
NSUB = C // G
NP = 8
VW = DV
N2 = 10112
RPT = N2 // NS
ZR = 8
MB = 400


def _proj_body(x_ref, wk_ref, bk_ref, wq_ref, bq_ref, wv_ref, bv_ref,
               k_ref, q_ref, *v_refs):
    x = x_ref[...]
    dn = (((1,), (1,)), ((), ()))
    k_ref[...] = lax.dot_general(x, wk_ref[...], dn,
                                 preferred_element_type=jnp.float32) + bk_ref[...]
    q_ref[...] = lax.dot_general(x, wq_ref[...], dn,
                                 preferred_element_type=jnp.float32) + bq_ref[...]
    v = lax.dot_general(x, wv_ref[...], dn,
                        preferred_element_type=jnp.float32) + bv_ref[...]
    for h in range(H):
        v_refs[h][...] = v[:, h * VW:(h + 1) * VW]


def _proj(x, Wk, bk, Wq, bq, Wv, bv):
    nblk = N // MB
    wspec = pl.BlockSpec((HDK, D), lambda i: (0, 0))
    bspec = pl.BlockSpec((1, HDK), lambda i: (0, 0))
    rspec = pl.BlockSpec((MB, HDK), lambda i: (i, 0))
    vspec = pl.BlockSpec((MB, VW), lambda i: (i, 0))
    return pl.pallas_call(
        _proj_body,
        grid=(nblk,),
        in_specs=[pl.BlockSpec((MB, D), lambda i: (i, 0)),
                  wspec, bspec, wspec, bspec, wspec, bspec],
        out_specs=[rspec, rspec] + [vspec] * H,
        out_shape=[jax.ShapeDtypeStruct((N, HDK), jnp.float32),
                   jax.ShapeDtypeStruct((N, HDK), jnp.float32)]
                  + [jax.ShapeDtypeStruct((N, VW), jnp.float32)] * H,
    )(x, Wk, bk.reshape(1, -1), Wq, bq.reshape(1, -1), Wv, bv.reshape(1, -1))


def _sc_body(ei_hbm, k_hbm, q_hbm, *rest):
    v_hbm = list(rest[:H])
    o_flat = list(rest[H:H + NC * NP])
    ss0, ss1 = rest[H + NC * NP:H + NC * NP + NC]
    (ridx, sidx, att, qbuf0, qbuf1, kbuf0, kbuf1, vbuf0, vbuf1,
     mbuf0, mbuf1, attp0, attp1, zacc, sq0, sq1, sk0, sk1, sa0, sa1,
     acc_ref, ssum_ref) = rest[H + NC * NP + NC:]
    o_hbm = [o_flat[:NP], o_flat[NP:]]
    qb = [qbuf0, qbuf1]
    kb = [kbuf0, kbuf1]
    vb = [vbuf0, vbuf1]
    mb = [mbuf0, mbuf1]
    ap = [attp0, attp1]
    sq = [sq0, sq1]
    sk = [sk0, sk1]
    sa = [sa0, sa1]
    cid = lax.axis_index("c")
    sid = lax.axis_index("s")
    wid = cid * NS + sid
    row0 = sid * RPT

    pltpu.sync_copy(ei_hbm.at[0, wid], ridx)
    pltpu.sync_copy(ei_hbm.at[1, wid], sidx)

    for b in range(2):
        pltpu.async_copy(q_hbm.at[ridx.at[b]], qb[b], sq[b])
        pltpu.async_copy(k_hbm.at[sidx.at[b]], kb[b], sk[b])

    zv = jnp.zeros((L,), jnp.float32)

    def _z1(i, _):
        zacc[i // (VW // L), pl.ds((i % (VW // L)) * L, L)] = zv
        return 0
    lax.fori_loop(0, ZR * (VW // L), _z1, 0)

    for r in range(RPT // ZR):
        pltpu.sync_copy(zacc.at[:, pl.ds(0, L)],
                        ssum_ref.at[pl.ds(row0 + r * ZR, ZR)])
    plsc.subcore_barrier()

    lane = lax.iota(jnp.int32, L)
    hsel = jnp.minimum(lane, H)
    perms = [jnp.bitwise_xor(lane, sh) for sh in (8, 4, 2, 1)]

    gd = lax.GatherDimensionNumbers(offset_dims=(), collapsed_slice_dims=(0,),
                                    start_index_map=(0,))

    def _shuf(v, pm):
        return lax.gather(v, pm[:, None], gd, (1,),
                          mode=lax.GatherScatterMode.PROMISE_IN_BOUNDS)

    def _lanesum(v):
        for pm in perms:
            v = v + _shuf(v, pm)
        return v

    for b in range(2):
        for g in range(G):
            ap[b][g, :] = zv
        pltpu.async_copy(ap[b], ssum_ref.at[ridx.at[b]], sa[b], add=True)

    def _p1(jj, _):
        for b in range(2):
            j = 2 * jj + b
            pltpu.make_async_copy(q_hbm.at[ridx.at[j]], qb[b], sq[b]).wait()
            pltpu.make_async_copy(k_hbm.at[sidx.at[j]], kb[b], sk[b]).wait()
            pltpu.make_async_copy(ap[b], ssum_ref.at[ridx.at[j]],
                                  sa[b]).wait()

            def _edge(g, _, b=b):
                e = j * G + g
                attv = jnp.zeros((L,), jnp.float32)
                for h in range(H):
                    s = (qb[b][g, pl.ds(h * DK, L)]
                         * kb[b][g, pl.ds(h * DK, L)])
                    for t in range(1, DK // L):
                        s = s + (qb[b][g, pl.ds(h * DK + t * L, L)]
                                 * kb[b][g, pl.ds(h * DK + t * L, L)])
                    attv = jnp.where(lane == h, _lanesum(s), attv)
                attv = jnp.exp(attv * 0.125)
                ev = jnp.full((L,), e, jnp.int32)
                plsc.store_scatter(att, [hsel, ev], attv, mask=lane < H)
                ap[b][g, :] = jnp.where(lane < H, attv, 0.0)
                return 0
            lax.fori_loop(0, G, _edge, 0)

            jm = jnp.minimum(j + 2, NSUB - 1)
            pltpu.async_copy(q_hbm.at[ridx.at[jm]], qb[b], sq[b])
            pltpu.async_copy(k_hbm.at[sidx.at[jm]], kb[b], sk[b])

            pltpu.async_copy(ap[b], ssum_ref.at[ridx.at[j]], sa[b], add=True)
        return 0
    lax.fori_loop(0, NSUB // 2, _p1, 0)
    for b in range(2):
        pltpu.make_async_copy(q_hbm.at[ridx.at[b]], qb[b], sq[b]).wait()
        pltpu.make_async_copy(k_hbm.at[sidx.at[b]], kb[b], sk[b]).wait()
        pltpu.make_async_copy(ap[b], ssum_ref.at[ridx.at[b]], sa[b]).wait()
    plsc.subcore_barrier()

    ss_out = [ss0, ss1]
    for c in range(NC):
        pl.when(cid == c)(lambda c=c: pltpu.sync_copy(
            ssum_ref.at[pl.ds(row0, RPT)], ss_out[c].at[pl.ds(row0, RPT)]))

    for p in range(NP):
        for b in range(2):
            pltpu.async_copy(v_hbm[p].at[sidx.at[b]], vb[b], sq[b])
        for r in range(RPT // ZR):
            pltpu.sync_copy(zacc, acc_ref.at[pl.ds(row0 + r * ZR, ZR)])
        for b in range(2):
            for g in range(G):
                for t in range(VW // L):
                    mb[b][g, pl.ds(t * L, L)] = zv
        plsc.subcore_barrier()
        for b in range(2):
            pltpu.async_copy(mb[b], acc_ref.at[ridx.at[b]], sa[b], add=True)

        def _p2(jj, _, p=p):
            for b in range(2):
                j = 2 * jj + b
                pltpu.make_async_copy(v_hbm[p].at[sidx.at[j]], vb[b],
                                      sq[b]).wait()
                def _edge(g, _, b=b):
                    e = j * G + g
                    a0 = jnp.full((L,), att[p, pl.ds(e, L)][0], jnp.float32)
                    for t in range(DV // L):
                        mb[b][g, pl.ds(t * L, L)] = (
                            a0 * vb[b][g, pl.ds(t * L, L)])
                    return 0
                lax.fori_loop(0, G, _edge, 0)

                jm = jnp.minimum(j + 2, NSUB - 1)
                pltpu.async_copy(v_hbm[p].at[sidx.at[jm]], vb[b], sq[b])

                pltpu.async_copy(mb[b], acc_ref.at[ridx.at[j]], sa[b],
                                 add=True)
            return 0
        lax.fori_loop(0, NSUB // 2, _p2, 0)
        for b in range(2):
            pltpu.make_async_copy(v_hbm[p].at[sidx.at[b]], vb[b],
                                  sq[b]).wait()
            pltpu.make_async_copy(mb[b], acc_ref.at[ridx.at[b]],
                                  sa[b]).wait()
        plsc.subcore_barrier()

        for c in range(NC):
            pl.when(cid == c)(lambda c=c, p=p: pltpu.sync_copy(
                acc_ref.at[pl.ds(row0, RPT)],
                o_hbm[c][p].at[pl.ds(row0, RPT)]))
        plsc.subcore_barrier()


def _sc_edge_pass(ei4, K, Q, V):
    mesh = plsc.VectorSubcoreMesh(core_axis_name="c", subcore_axis_name="s")
    out_type = ([jax.ShapeDtypeStruct((N2, VW), jnp.float32)] * (NC * NP)
                + [jax.ShapeDtypeStruct((N2, L), jnp.float32)] * NC)
    scratch = [
        pltpu.VMEM((NSUB, G), jnp.int32),
        pltpu.VMEM((NSUB, G), jnp.int32),
        pltpu.VMEM((H, C + L), jnp.float32),
        pltpu.VMEM((G, HDK), jnp.float32),
        pltpu.VMEM((G, HDK), jnp.float32),
        pltpu.VMEM((G, HDK), jnp.float32),
        pltpu.VMEM((G, HDK), jnp.float32),
        pltpu.VMEM((G, VW), jnp.float32),
        pltpu.VMEM((G, VW), jnp.float32),
        pltpu.VMEM((G, VW), jnp.float32),
        pltpu.VMEM((G, VW), jnp.float32),
        pltpu.VMEM((G, L), jnp.float32),
        pltpu.VMEM((G, L), jnp.float32),
        pltpu.VMEM((ZR, VW), jnp.float32),
        pltpu.SemaphoreType.DMA,
        pltpu.SemaphoreType.DMA,
        pltpu.SemaphoreType.DMA,
        pltpu.SemaphoreType.DMA,
        pltpu.SemaphoreType.DMA,
        pltpu.SemaphoreType.DMA,
        pltpu.VMEM_SHARED((N2, VW), jnp.float32),
        pltpu.VMEM_SHARED((N2, L), jnp.float32),
    ]
    f = pl.kernel(_sc_body, out_type=out_type, mesh=mesh,
                  scratch_types=scratch,
                  compiler_params=pltpu.CompilerParams(
                      use_tc_tiling_on_sc=False,
                      needs_layout_passes=False))
    return f(ei4, K, Q, *V)


def _final_body(x_ref, *rest):
    parts = rest[:NC * NP]
    (s0_ref, s1_ref, wagg_ref, bagg_ref, wff_ref, bff_ref, o_ref) = \
        rest[NC * NP:]
    a = jnp.concatenate(
        [parts[p][...] + parts[NP + p][...] for p in range(NP)], axis=1)
    ss = (s0_ref[...] + s1_ref[...])[:, :H]
    ss = jnp.where(ss == 0.0, 1.0, ss)
    den = jnp.broadcast_to(ss[:, :, None], (MB, H, DV)).reshape(MB, H * DV)
    a = jnp.maximum(a / den, 0.0)
    dn = (((1,), (1,)), ((), ()))
    t = lax.dot_general(a, wagg_ref[...], dn,
                        preferred_element_type=jnp.float32) + bagg_ref[...]
    t = jnp.maximum(t, 0.0)
    y = lax.dot_general(t, wff_ref[...], dn,
                        preferred_element_type=jnp.float32) + bff_ref[...]
    y = jnp.maximum(y, 0.0)
    o_ref[...] = x_ref[...] + y


def _final(x, parts, ssums, Wagg, bagg, Wff, bff):
    nblk = N // MB
    pspec = pl.BlockSpec((MB, VW), lambda i: (i, 0))
    sspec = pl.BlockSpec((MB, L), lambda i: (i, 0))
    return pl.pallas_call(
        _final_body,
        grid=(nblk,),
        in_specs=[pl.BlockSpec((MB, D), lambda i: (i, 0))]
                 + [pspec] * (NC * NP) + [sspec] * NC
                 + [pl.BlockSpec((D, H * DV), lambda i: (0, 0)),
                    pl.BlockSpec((1, D), lambda i: (0, 0)),
                    pl.BlockSpec((D, D), lambda i: (0, 0)),
                    pl.BlockSpec((1, D), lambda i: (0, 0))],
        out_specs=pl.BlockSpec((MB, D), lambda i: (i, 0)),
        out_shape=jax.ShapeDtypeStruct((N, D), jnp.float32),
    )(x, *parts, *ssums, Wagg, bagg.reshape(1, -1), Wff, bff.reshape(1, -1))


def kernel(x, edge_index, Wk, bk, Wq, bq, Wv, bv, Wagg, bagg, Wff, bff):
    K, Q, *Vs = _proj(x, Wk, bk, Wq, bq, Wv, bv)
    ei4 = edge_index.reshape(2, NW, NSUB, G)
    outs = _sc_edge_pass(ei4, K, Q, Vs)
    parts, ssums = outs[:NC * NP], outs[NC * NP:]
    return _final(x, parts, ssums, Wagg, bagg, Wff, bff)


# speedup vs baseline: 1.3067x; 1.3067x over previous
import functools

import jax
import jax.numpy as jnp
from jax import lax
from jax.experimental import pallas as pl
from jax.experimental.pallas import tpu as pltpu
from jax.experimental.pallas import tpu_sc as plsc

N, E, D, H, DK, DV = 10000, 160000, 256, 8, 64, 64
HDK = H * DK
NC, NS, L = 2, 16, 16
NW = NC * NS
C = E // NW
G = 10
NSUB = C // G
NP = 8
VW = DV
N2 = 10112
RPT = N2 // NS
ZR = 8
MB = 400


def _proj_body(x_ref, wk_ref, bk_ref, wq_ref, bq_ref, wv_ref, bv_ref,
               k_ref, q_ref, *v_refs):
    x = x_ref[...]
    dn = (((1,), (1,)), ((), ()))
    k_ref[...] = (lax.dot_general(x, wk_ref[...], dn,
                                  preferred_element_type=jnp.float32)
                  + bk_ref[...]).astype(jnp.bfloat16)
    q_ref[...] = (lax.dot_general(x, wq_ref[...], dn,
                                  preferred_element_type=jnp.float32)
                  + bq_ref[...]).astype(jnp.bfloat16)
    v = lax.dot_general(x, wv_ref[...], dn,
                        preferred_element_type=jnp.float32) + bv_ref[...]
    for h in range(H):
        v_refs[h][...] = v[:, h * VW:(h + 1) * VW]


def _proj(x, Wk, bk, Wq, bq, Wv, bv):
    nblk = N // MB
    wspec = pl.BlockSpec((HDK, D), lambda i: (0, 0))
    bspec = pl.BlockSpec((1, HDK), lambda i: (0, 0))
    rspec = pl.BlockSpec((MB, HDK), lambda i: (i, 0))
    vspec = pl.BlockSpec((MB, VW), lambda i: (i, 0))
    return pl.pallas_call(
        _proj_body,
        grid=(nblk,),
        in_specs=[pl.BlockSpec((MB, D), lambda i: (i, 0)),
                  wspec, bspec, wspec, bspec, wspec, bspec],
        out_specs=[rspec, rspec] + [vspec] * H,
        out_shape=[jax.ShapeDtypeStruct((N, HDK), jnp.bfloat16),
                   jax.ShapeDtypeStruct((N, HDK), jnp.bfloat16)]
                  + [jax.ShapeDtypeStruct((N, VW), jnp.float32)] * H,
    )(x, Wk, bk.reshape(1, -1), Wq, bq.reshape(1, -1), Wv, bv.reshape(1, -1))


def _sc_body(ei_hbm, k_hbm, q_hbm, *rest):
    v_hbm = list(rest[:H])
    o_flat = list(rest[H:H + NC * NP])
    ss0, ss1 = rest[H + NC * NP:H + NC * NP + NC]
    (ridx, sidx, att, qbuf0, qbuf1, kbuf0, kbuf1,
     vbuf0, vbuf1, vbuf2, vbuf3, mbuf0, mbuf1, attp0, attp1, zacc,
     sq0, sq1, sq2, sq3, sk0, sk1, sa0, sa1,
     acc_ref, ssum_ref) = rest[H + NC * NP + NC:]
    o_hbm = [o_flat[:NP], o_flat[NP:]]
    qb = [qbuf0, qbuf1]
    kb = [kbuf0, kbuf1]
    vb = [vbuf0, vbuf1, vbuf2, vbuf3]
    mb = [mbuf0, mbuf1]
    ap = [attp0, attp1]
    sq = [sq0, sq1, sq2, sq3]
    sk = [sk0, sk1]
    sa = [sa0, sa1]
    cid = lax.axis_index("c")
    sid = lax.axis_index("s")
    wid = cid * NS + sid
    row0 = sid * RPT

    pltpu.sync_copy(ei_hbm.at[0, wid], ridx)
    pltpu.sync_copy(ei_hbm.at[1, wid], sidx)

    for b in range(2):
        pltpu.async_copy(q_hbm.at[ridx.at[b]], qb[b], sq[b])
        pltpu.async_copy(k_hbm.at[sidx.at[b]], kb[b], sk[b])

    zv = jnp.zeros((L,), jnp.float32)

    def _z1(i, _):
        zacc[i // (VW // L), pl.ds((i % (VW // L)) * L, L)] = zv
        return 0
    lax.fori_loop(0, ZR * (VW // L), _z1, 0)

    for r in range(RPT // ZR):
        pltpu.sync_copy(zacc.at[:, pl.ds(0, L)],
                        ssum_ref.at[pl.ds(row0 + r * ZR, ZR)])
    plsc.subcore_barrier()

    lane = lax.iota(jnp.int32, L)
    hsel = jnp.minimum(lane, H)
    perms = [jnp.bitwise_xor(lane, sh) for sh in (8, 4, 2, 1)]

    gd = lax.GatherDimensionNumbers(offset_dims=(), collapsed_slice_dims=(0,),
                                    start_index_map=(0,))

    def _shuf(v, pm):
        return lax.gather(v, pm[:, None], gd, (1,),
                          mode=lax.GatherScatterMode.PROMISE_IN_BOUNDS)

    def _lanesum(v):
        for pm in perms:
            v = v + _shuf(v, pm)
        return v

    def _p1(jj, _):
        for b in range(2):
            j = 2 * jj + b
            pltpu.make_async_copy(q_hbm.at[ridx.at[j]], qb[b], sq[b]).wait()
            pltpu.make_async_copy(k_hbm.at[sidx.at[j]], kb[b], sk[b]).wait()

            def _edge(g, _, b=b):
                e = j * G + g
                attv = jnp.zeros((L,), jnp.float32)
                for h in range(H):
                    s = jnp.zeros((L,), jnp.float32)
                    for t in range(DK // (2 * L)):
                        qv = qb[b][g, pl.ds(h * DK + t * 2 * L, 2 * L)]
                        kv = kb[b][g, pl.ds(h * DK + t * 2 * L, 2 * L)]
                        qa, qc = plsc.unpack(qv,
                                             format=plsc.PackFormat.INTERLEAVED)
                        ka, kc = plsc.unpack(kv,
                                             format=plsc.PackFormat.INTERLEAVED)
                        s = s + qa * ka + qc * kc
                    attv = jnp.where(lane == h, _lanesum(s), attv)
                attv = jnp.exp(attv * 0.125)
                ev = jnp.full((L,), e, jnp.int32)
                plsc.store_scatter(att, [hsel, ev], attv, mask=lane < H)
                ap[b][g, :] = jnp.where(lane < H, attv, 0.0)
                return 0
            lax.fori_loop(0, G, _edge, 0)

            pl.when(j + 2 < NSUB)(lambda j=j, b=b: (
                pltpu.async_copy(q_hbm.at[ridx.at[j + 2]], qb[b], sq[b]),
                pltpu.async_copy(k_hbm.at[sidx.at[j + 2]], kb[b], sk[b]),
                None)[-1])

            pltpu.sync_copy(ap[b], ssum_ref.at[ridx.at[j]], add=True)
        return 0
    lax.fori_loop(0, NSUB // 2, _p1, 0)
    plsc.subcore_barrier()

    ss_out = [ss0, ss1]
    for c in range(NC):
        pl.when(cid == c)(lambda c=c: pltpu.sync_copy(
            ssum_ref.at[pl.ds(row0, RPT)], ss_out[c].at[pl.ds(row0, RPT)]))

    NB = 4
    for p in range(NP):
        for b in range(NB):
            pltpu.async_copy(v_hbm[p].at[sidx.at[b]], vb[b], sq[b])
        for r in range(RPT // ZR):
            pltpu.sync_copy(zacc, acc_ref.at[pl.ds(row0 + r * ZR, ZR)])
        plsc.subcore_barrier()

        def _p2(jj, _, p=p):
            for b in range(NB):
                j = NB * jj + b
                pltpu.make_async_copy(v_hbm[p].at[sidx.at[j]], vb[b],
                                      sq[b]).wait()
                def _edge(g, _, b=b):
                    e = j * G + g
                    a0 = jnp.full((L,), att[p, pl.ds(e, L)][0], jnp.float32)
                    for t in range(DV // L):
                        mb[b % 2][g, pl.ds(t * L, L)] = (
                            a0 * vb[b][g, pl.ds(t * L, L)])
                    return 0
                lax.fori_loop(0, G, _edge, 0)

                jm = jnp.minimum(j + NB, NSUB - 1)
                pltpu.async_copy(v_hbm[p].at[sidx.at[jm]], vb[b], sq[b])

                pltpu.sync_copy(mb[b % 2], acc_ref.at[ridx.at[j]], add=True)
            return 0
        lax.fori_loop(0, NSUB // NB, _p2, 0)
        for b in range(NB):
            pltpu.make_async_copy(v_hbm[p].at[sidx.at[b]], vb[b],
                                  sq[b]).wait()
        plsc.subcore_barrier()

        for c in range(NC):
            pl.when(cid == c)(lambda c=c, p=p: pltpu.sync_copy(
                acc_ref.at[pl.ds(row0, RPT)],
                o_hbm[c][p].at[pl.ds(row0, RPT)]))
        plsc.subcore_barrier()


def _sc_edge_pass(ei4, K, Q, V):
    mesh = plsc.VectorSubcoreMesh(core_axis_name="c", subcore_axis_name="s")
    out_type = ([jax.ShapeDtypeStruct((N2, VW), jnp.float32)] * (NC * NP)
                + [jax.ShapeDtypeStruct((N2, L), jnp.float32)] * NC)
    scratch = [
        pltpu.VMEM((NSUB, G), jnp.int32),
        pltpu.VMEM((NSUB, G), jnp.int32),
        pltpu.VMEM((H, C + L), jnp.float32),
        pltpu.VMEM((G, HDK), jnp.bfloat16),
        pltpu.VMEM((G, HDK), jnp.bfloat16),
        pltpu.VMEM((G, HDK), jnp.bfloat16),
        pltpu.VMEM((G, HDK), jnp.bfloat16),
        pltpu.VMEM((G, VW), jnp.float32),
        pltpu.VMEM((G, VW), jnp.float32),
        pltpu.VMEM((G, VW), jnp.float32),
        pltpu.VMEM((G, VW), jnp.float32),
        pltpu.VMEM((G, VW), jnp.float32),
        pltpu.VMEM((G, VW), jnp.float32),
        pltpu.VMEM((G, L), jnp.float32),
        pltpu.VMEM((G, L), jnp.float32),
        pltpu.VMEM((ZR, VW), jnp.float32),
        pltpu.SemaphoreType.DMA,
        pltpu.SemaphoreType.DMA,
        pltpu.SemaphoreType.DMA,
        pltpu.SemaphoreType.DMA,
        pltpu.SemaphoreType.DMA,
        pltpu.SemaphoreType.DMA,
        pltpu.SemaphoreType.DMA,
        pltpu.SemaphoreType.DMA,
        pltpu.VMEM_SHARED((N2, VW), jnp.float32),
        pltpu.VMEM_SHARED((N2, L), jnp.float32),
    ]
    f = pl.kernel(_sc_body, out_type=out_type, mesh=mesh,
                  scratch_types=scratch,
                  compiler_params=pltpu.CompilerParams(
                      use_tc_tiling_on_sc=False,
                      needs_layout_passes=False))
    return f(ei4, K, Q, *V)


def _final_body(x_ref, *rest):
    parts = rest[:NC * NP]
    (s0_ref, s1_ref, wagg_ref, bagg_ref, wff_ref, bff_ref, o_ref) = \
        rest[NC * NP:]
    a = jnp.concatenate(
        [parts[p][...] + parts[NP + p][...] for p in range(NP)], axis=1)
    ss = (s0_ref[...] + s1_ref[...])[:, :H]
    ss = jnp.where(ss == 0.0, 1.0, ss)
    den = jnp.broadcast_to(ss[:, :, None], (MB, H, DV)).reshape(MB, H * DV)
    a = jnp.maximum(a / den, 0.0)
    dn = (((1,), (1,)), ((), ()))
    t = lax.dot_general(a, wagg_ref[...], dn,
                        preferred_element_type=jnp.float32) + bagg_ref[...]
    t = jnp.maximum(t, 0.0)
    y = lax.dot_general(t, wff_ref[...], dn,
                        preferred_element_type=jnp.float32) + bff_ref[...]
    y = jnp.maximum(y, 0.0)
    o_ref[...] = x_ref[...] + y


def _final(x, parts, ssums, Wagg, bagg, Wff, bff):
    nblk = N // MB
    pspec = pl.BlockSpec((MB, VW), lambda i: (i, 0))
    sspec = pl.BlockSpec((MB, L), lambda i: (i, 0))
    return pl.pallas_call(
        _final_body,
        grid=(nblk,),
        in_specs=[pl.BlockSpec((MB, D), lambda i: (i, 0))]
                 + [pspec] * (NC * NP) + [sspec] * NC
                 + [pl.BlockSpec((D, H * DV), lambda i: (0, 0)),
                    pl.BlockSpec((1, D), lambda i: (0, 0)),
                    pl.BlockSpec((D, D), lambda i: (0, 0)),
                    pl.BlockSpec((1, D), lambda i: (0, 0))],
        out_specs=pl.BlockSpec((MB, D), lambda i: (i, 0)),
        out_shape=jax.ShapeDtypeStruct((N, D), jnp.float32),
    )(x, *parts, *ssums, Wagg, bagg.reshape(1, -1), Wff, bff.reshape(1, -1))


def kernel(x, edge_index, Wk, bk, Wq, bq, Wv, bv, Wagg, bagg, Wff, bff):
    K, Q, *Vs = _proj(x, Wk, bk, Wq, bq, Wv, bv)
    ei4 = edge_index.reshape(2, NW, NSUB, G)
    outs = _sc_edge_pass(ei4, K, Q, Vs)
    parts, ssums = outs[:NC * NP], outs[NC * NP:]
    return _final(x, parts, ssums, Wagg, bagg, Wff, bff)


# speedup vs baseline: 1.3352x; 1.0218x over previous
import functools

import jax
import jax.numpy as jnp
from jax import lax
from jax.experimental import pallas as pl
from jax.experimental.pallas import tpu as pltpu
from jax.experimental.pallas import tpu_sc as plsc

N, E, D, H, DK, DV = 10000, 160000, 256, 8, 64, 64
HDK = H * DK
NC, NS, L = 2, 16, 16
NW = NC * NS
C = E // NW
G = 10
NSUB = C // G
NP = 8
VW = DV
N2 = 10112
RPT = N2 // NS
ZR = 8
MB = 400


def _proj_body(x_ref, wk_ref, bk_ref, wq_ref, bq_ref, wv_ref, bv_ref,
               k_ref, q_ref, *v_refs):
    x = x_ref[...]
    dn = (((1,), (1,)), ((), ()))
    k_ref[...] = (lax.dot_general(x, wk_ref[...], dn,
                                  preferred_element_type=jnp.float32)
                  + bk_ref[...]).astype(jnp.bfloat16)
    q_ref[...] = (lax.dot_general(x, wq_ref[...], dn,
                                  preferred_element_type=jnp.float32)
                  + bq_ref[...]).astype(jnp.bfloat16)
    v = lax.dot_general(x, wv_ref[...], dn,
                        preferred_element_type=jnp.float32) + bv_ref[...]
    for h in range(H):
        v_refs[h][...] = v[:, h * VW:(h + 1) * VW]


def _proj(x, Wk, bk, Wq, bq, Wv, bv):
    nblk = N // MB
    wspec = pl.BlockSpec((HDK, D), lambda i: (0, 0))
    bspec = pl.BlockSpec((1, HDK), lambda i: (0, 0))
    rspec = pl.BlockSpec((MB, HDK), lambda i: (i, 0))
    vspec = pl.BlockSpec((MB, VW), lambda i: (i, 0))
    return pl.pallas_call(
        _proj_body,
        grid=(nblk,),
        in_specs=[pl.BlockSpec((MB, D), lambda i: (i, 0)),
                  wspec, bspec, wspec, bspec, wspec, bspec],
        out_specs=[rspec, rspec] + [vspec] * H,
        out_shape=[jax.ShapeDtypeStruct((N, HDK), jnp.bfloat16),
                   jax.ShapeDtypeStruct((N, HDK), jnp.bfloat16)]
                  + [jax.ShapeDtypeStruct((N, VW), jnp.float32)] * H,
    )(x, Wk, bk.reshape(1, -1), Wq, bq.reshape(1, -1), Wv, bv.reshape(1, -1))


def _sc_body(ei_hbm, k_hbm, q_hbm, *rest):
    v_hbm = list(rest[:H])
    o_flat = list(rest[H:H + NC * NP])
    ss0, ss1 = rest[H + NC * NP:H + NC * NP + NC]
    (ridx, sidx, att, qbuf0, qbuf1, kbuf0, kbuf1,
     vbuf0, vbuf1, vbuf2, vbuf3, mbuf0, mbuf1, mbuf2, mbuf3,
     attp0, attp1, zacc,
     sq0, sq1, sq2, sq3, sk0, sk1, sa0, sa1, sa2, sa3, sz,
     acc_ref, ssum_ref) = rest[H + NC * NP + NC:]
    o_hbm = [o_flat[:NP], o_flat[NP:]]
    qb = [qbuf0, qbuf1]
    kb = [kbuf0, kbuf1]
    vb = [vbuf0, vbuf1, vbuf2, vbuf3]
    mb = [mbuf0, mbuf1, mbuf2, mbuf3]
    ap = [attp0, attp1]
    sq = [sq0, sq1, sq2, sq3]
    sk = [sk0, sk1]
    sa = [sa0, sa1, sa2, sa3]
    cid = lax.axis_index("c")
    sid = lax.axis_index("s")
    wid = cid * NS + sid
    row0 = sid * RPT

    pltpu.sync_copy(ei_hbm.at[0, wid], ridx)
    pltpu.sync_copy(ei_hbm.at[1, wid], sidx)

    for b in range(2):
        pltpu.async_copy(q_hbm.at[ridx.at[b]], qb[b], sq[b])
        pltpu.async_copy(k_hbm.at[sidx.at[b]], kb[b], sk[b])

    zv = jnp.zeros((L,), jnp.float32)

    def _z1(i, _):
        zacc[i // (VW // L), pl.ds((i % (VW // L)) * L, L)] = zv
        return 0
    lax.fori_loop(0, ZR * (VW // L), _z1, 0)

    for r in range(RPT // ZR):
        pltpu.sync_copy(zacc.at[:, pl.ds(0, L)],
                        ssum_ref.at[pl.ds(row0 + r * ZR, ZR)])
    plsc.subcore_barrier()

    lane = lax.iota(jnp.int32, L)
    hsel = jnp.minimum(lane, H)
    perms = [jnp.bitwise_xor(lane, sh) for sh in (8, 4, 2, 1)]

    gd = lax.GatherDimensionNumbers(offset_dims=(), collapsed_slice_dims=(0,),
                                    start_index_map=(0,))

    def _shuf(v, pm):
        return lax.gather(v, pm[:, None], gd, (1,),
                          mode=lax.GatherScatterMode.PROMISE_IN_BOUNDS)

    def _lanesum(v):
        for pm in perms:
            v = v + _shuf(v, pm)
        return v

    def _p1(jj, _):
        for b in range(2):
            j = 2 * jj + b
            pltpu.make_async_copy(q_hbm.at[ridx.at[j]], qb[b], sq[b]).wait()
            pltpu.make_async_copy(k_hbm.at[sidx.at[j]], kb[b], sk[b]).wait()

            def _edge(g, _, b=b):
                e = j * G + g
                attv = jnp.zeros((L,), jnp.float32)
                for h in range(H):
                    s = jnp.zeros((L,), jnp.float32)
                    for t in range(DK // (2 * L)):
                        qv = qb[b][g, pl.ds(h * DK + t * 2 * L, 2 * L)]
                        kv = kb[b][g, pl.ds(h * DK + t * 2 * L, 2 * L)]
                        qa, qc = plsc.unpack(qv,
                                             format=plsc.PackFormat.INTERLEAVED)
                        ka, kc = plsc.unpack(kv,
                                             format=plsc.PackFormat.INTERLEAVED)
                        s = s + qa * ka + qc * kc
                    attv = jnp.where(lane == h, _lanesum(s), attv)
                attv = jnp.exp(attv * 0.125)
                ev = jnp.full((L,), e, jnp.int32)
                plsc.store_scatter(att, [hsel, ev], attv, mask=lane < H)
                ap[b][g, :] = jnp.where(lane < H, attv, 0.0)
                return 0
            lax.fori_loop(0, G, _edge, 0)

            pl.when(j + 2 < NSUB)(lambda j=j, b=b: (
                pltpu.async_copy(q_hbm.at[ridx.at[j + 2]], qb[b], sq[b]),
                pltpu.async_copy(k_hbm.at[sidx.at[j + 2]], kb[b], sk[b]),
                None)[-1])

            pltpu.sync_copy(ap[b], ssum_ref.at[ridx.at[j]], add=True)
        return 0
    lax.fori_loop(0, NSUB // 2, _p1, 0)
    plsc.subcore_barrier()

    ss_out = [ss0, ss1]
    for c in range(NC):
        pl.when(cid == c)(lambda c=c: pltpu.sync_copy(
            ssum_ref.at[pl.ds(row0, RPT)], ss_out[c].at[pl.ds(row0, RPT)]))

    NB = 4
    for p in range(NP):
        for b in range(NB):
            pltpu.async_copy(v_hbm[p].at[sidx.at[b]], vb[b], sq[b])
        for r in range(RPT // ZR):
            pltpu.async_copy(zacc, acc_ref.at[pl.ds(row0 + r * ZR, ZR)], sz)
        for r in range(RPT // ZR):
            pltpu.make_async_copy(
                zacc, acc_ref.at[pl.ds(row0 + r * ZR, ZR)], sz).wait()
        plsc.subcore_barrier()

        def _p2(jj, _, p=p):
            for b in range(NB):
                j = NB * jj + b
                pltpu.make_async_copy(v_hbm[p].at[sidx.at[j]], vb[b],
                                      sq[b]).wait()
                def _edge(g, _, b=b):
                    e = j * G + g
                    a0 = jnp.full((L,), att[p, pl.ds(e, L)][0], jnp.float32)
                    for t in range(DV // L):
                        mb[b % 2][g, pl.ds(t * L, L)] = (
                            a0 * vb[b][g, pl.ds(t * L, L)])
                    return 0
                lax.fori_loop(0, G, _edge, 0)

                jm = jnp.minimum(j + NB, NSUB - 1)
                pltpu.async_copy(v_hbm[p].at[sidx.at[jm]], vb[b], sq[b])

                pltpu.sync_copy(mb[b % 2], acc_ref.at[ridx.at[j]], add=True)
            return 0
        lax.fori_loop(0, NSUB // NB, _p2, 0)
        for b in range(NB):
            pltpu.make_async_copy(v_hbm[p].at[sidx.at[b]], vb[b],
                                  sq[b]).wait()
        plsc.subcore_barrier()

        for c in range(NC):
            pl.when(cid == c)(lambda c=c, p=p: pltpu.sync_copy(
                acc_ref.at[pl.ds(row0, RPT)],
                o_hbm[c][p].at[pl.ds(row0, RPT)]))
        plsc.subcore_barrier()


def _sc_edge_pass(ei4, K, Q, V):
    mesh = plsc.VectorSubcoreMesh(core_axis_name="c", subcore_axis_name="s")
    out_type = ([jax.ShapeDtypeStruct((N2, VW), jnp.float32)] * (NC * NP)
                + [jax.ShapeDtypeStruct((N2, L), jnp.float32)] * NC)
    scratch = [
        pltpu.VMEM((NSUB, G), jnp.int32),
        pltpu.VMEM((NSUB, G), jnp.int32),
        pltpu.VMEM((H, C + L), jnp.float32),
        pltpu.VMEM((G, HDK), jnp.bfloat16),
        pltpu.VMEM((G, HDK), jnp.bfloat16),
        pltpu.VMEM((G, HDK), jnp.bfloat16),
        pltpu.VMEM((G, HDK), jnp.bfloat16),
        pltpu.VMEM((G, VW), jnp.float32),
        pltpu.VMEM((G, VW), jnp.float32),
        pltpu.VMEM((G, VW), jnp.float32),
        pltpu.VMEM((G, VW), jnp.float32),
        pltpu.VMEM((G, VW), jnp.float32),
        pltpu.VMEM((G, VW), jnp.float32),
        pltpu.VMEM((G, VW), jnp.float32),
        pltpu.VMEM((G, VW), jnp.float32),
        pltpu.VMEM((G, L), jnp.float32),
        pltpu.VMEM((G, L), jnp.float32),
        pltpu.VMEM((ZR, VW), jnp.float32),
        pltpu.SemaphoreType.DMA,
        pltpu.SemaphoreType.DMA,
        pltpu.SemaphoreType.DMA,
        pltpu.SemaphoreType.DMA,
        pltpu.SemaphoreType.DMA,
        pltpu.SemaphoreType.DMA,
        pltpu.SemaphoreType.DMA,
        pltpu.SemaphoreType.DMA,
        pltpu.SemaphoreType.DMA,
        pltpu.SemaphoreType.DMA,
        pltpu.SemaphoreType.DMA,
        pltpu.VMEM_SHARED((N2, VW), jnp.float32),
        pltpu.VMEM_SHARED((N2, L), jnp.float32),
    ]
    f = pl.kernel(_sc_body, out_type=out_type, mesh=mesh,
                  scratch_types=scratch,
                  compiler_params=pltpu.CompilerParams(
                      use_tc_tiling_on_sc=False,
                      needs_layout_passes=False))
    return f(ei4, K, Q, *V)


def _final_body(x_ref, *rest):
    parts = rest[:NC * NP]
    (s0_ref, s1_ref, wagg_ref, bagg_ref, wff_ref, bff_ref, o_ref) = \
        rest[NC * NP:]
    a = jnp.concatenate(
        [parts[p][...] + parts[NP + p][...] for p in range(NP)], axis=1)
    ss = (s0_ref[...] + s1_ref[...])[:, :H]
    ss = jnp.where(ss == 0.0, 1.0, ss)
    den = jnp.broadcast_to(ss[:, :, None], (MB, H, DV)).reshape(MB, H * DV)
    a = jnp.maximum(a / den, 0.0)
    dn = (((1,), (1,)), ((), ()))
    t = lax.dot_general(a, wagg_ref[...], dn,
                        preferred_element_type=jnp.float32) + bagg_ref[...]
    t = jnp.maximum(t, 0.0)
    y = lax.dot_general(t, wff_ref[...], dn,
                        preferred_element_type=jnp.float32) + bff_ref[...]
    y = jnp.maximum(y, 0.0)
    o_ref[...] = x_ref[...] + y


def _final(x, parts, ssums, Wagg, bagg, Wff, bff):
    nblk = N // MB
    pspec = pl.BlockSpec((MB, VW), lambda i: (i, 0))
    sspec = pl.BlockSpec((MB, L), lambda i: (i, 0))
    return pl.pallas_call(
        _final_body,
        grid=(nblk,),
        in_specs=[pl.BlockSpec((MB, D), lambda i: (i, 0))]
                 + [pspec] * (NC * NP) + [sspec] * NC
                 + [pl.BlockSpec((D, H * DV), lambda i: (0, 0)),
                    pl.BlockSpec((1, D), lambda i: (0, 0)),
                    pl.BlockSpec((D, D), lambda i: (0, 0)),
                    pl.BlockSpec((1, D), lambda i: (0, 0))],
        out_specs=pl.BlockSpec((MB, D), lambda i: (i, 0)),
        out_shape=jax.ShapeDtypeStruct((N, D), jnp.float32),
    )(x, *parts, *ssums, Wagg, bagg.reshape(1, -1), Wff, bff.reshape(1, -1))


def kernel(x, edge_index, Wk, bk, Wq, bq, Wv, bv, Wagg, bagg, Wff, bff):
    K, Q, *Vs = _proj(x, Wk, bk, Wq, bq, Wv, bv)
    ei4 = edge_index.reshape(2, NW, NSUB, G)
    outs = _sc_edge_pass(ei4, K, Q, Vs)
    parts, ssums = outs[:NC * NP], outs[NC * NP:]
    return _final(x, parts, ssums, Wagg, bagg, Wff, bff)
